# Initial kernel scaffold; baseline (speedup 1.0000x reference)
import jax
import jax.numpy as jnp
from jax.experimental import pallas as pl


def kernel(x, W_gate, W1, W2, k):
    raise NotImplementedError("write your pallas kernel here")


# speedup vs baseline: 1.5687x; 1.5687x over previous
import functools
import math

import jax
import jax.numpy as jnp
from jax import lax
from jax.experimental import pallas as pl
from jax.experimental.pallas import tpu as pltpu
from jax.experimental.pallas import tpu_sc as plsc

CAPACITY_FACTOR = 1.25
K_TOP = 2


def _route_body(scale_ref, x_ref, wg_ref,
                slot0_ref, slot1_ref, w0b_ref, w1b_ref, aux_ref,
                *, T, E, C, CP, BLK):
    scale = scale_ref[0, 0]
    logits = jnp.dot(x_ref[...], wg_ref[...],
                     preferred_element_type=jnp.float32) * scale

    iota_e = lax.broadcasted_iota(jnp.int32, (T, E), 1)
    m1 = jnp.max(logits, axis=1, keepdims=True)
    a1 = jnp.min(jnp.where(logits == m1, iota_e, E), axis=1, keepdims=True)
    oh0 = (iota_e == a1)
    l2 = jnp.where(oh0, jnp.float32(-1e30), logits)
    m2 = jnp.max(l2, axis=1, keepdims=True)
    a2 = jnp.min(jnp.where(l2 == m2, iota_e, E), axis=1, keepdims=True)
    oh1 = (iota_e == a2)

    z = jnp.exp(m2 - m1)
    p0 = 1.0 / (1.0 + z)
    p1 = z / (1.0 + z)

    oh0f = oh0.astype(jnp.float32)
    oh1f = oh1.astype(jnp.float32)
    ohsum = oh0f + oh1f

    nb = T // BLK
    tril = (lax.broadcasted_iota(jnp.int32, (BLK, BLK), 0)
            > lax.broadcasted_iota(jnp.int32, (BLK, BLK), 1)).astype(jnp.float32)
    carry = jnp.zeros((1, E), jnp.float32)
    pieces = []
    for b in range(nb):
        blk = ohsum[b * BLK:(b + 1) * BLK, :]
        local = jnp.dot(tril, blk, preferred_element_type=jnp.float32)
        pieces.append(local + carry)
        carry = carry + jnp.sum(blk, axis=0, keepdims=True)
    cumexcl = jnp.concatenate(pieces, axis=0)

    cnt0 = jnp.sum(oh0f * cumexcl, axis=1, keepdims=True) + 1.0
    cnt1 = jnp.sum(oh1f * cumexcl, axis=1, keepdims=True) + 1.0
    keep0 = cnt0 <= C
    keep1 = cnt1 <= C
    pos0 = cnt0.astype(jnp.int32) - 1
    pos1 = cnt1.astype(jnp.int32) - 1

    p0k = p0 * keep0.astype(jnp.float32)
    p1k = p1 * keep1.astype(jnp.float32)
    denom = p0k + p1k + 1e-9
    w0 = jnp.where(keep0, p0k / denom, 0.0)
    w1 = jnp.where(keep1, p1k / denom, 0.0)

    slot0 = jnp.where(keep0, a1 * CP + pos0, a1 * CP + C)
    slot1 = jnp.where(keep1, a2 * CP + pos1, a2 * CP + C)

    slot0_ref[...] = slot0[:, 0]
    slot1_ref[...] = slot1[:, 0]
    w0b_ref[...] = jnp.broadcast_to(w0, (T, 16))
    w1b_ref[...] = jnp.broadcast_to(w1, (T, 16))

    k0f = keep0.astype(jnp.float32)
    k1f = keep1.astype(jnp.float32)
    tokens_per_e = jnp.sum(k0f * oh0f + k1f * oh1f, axis=0, keepdims=True)
    importance = jnp.sum(w0 * oh0f + w1 * oh1f, axis=0, keepdims=True)
    tf = tokens_per_e / (jnp.sum(tokens_per_e) + 1e-9)
    imf = importance / (jnp.sum(importance) + 1e-9)
    aux_ref[0, 0] = jnp.sum(tf * imf) * E


def _route(xf, W_gate, scale, *, T, E, C, CP, interpret=False):
    body = functools.partial(_route_body, T=T, E=E, C=C, CP=CP, BLK=128)
    return pl.pallas_call(
        body,
        in_specs=[
            pl.BlockSpec(memory_space=pltpu.SMEM),
            pl.BlockSpec(memory_space=pltpu.VMEM),
            pl.BlockSpec(memory_space=pltpu.VMEM),
        ],
        out_specs=[
            pl.BlockSpec(memory_space=pltpu.VMEM),
            pl.BlockSpec(memory_space=pltpu.VMEM),
            pl.BlockSpec(memory_space=pltpu.VMEM),
            pl.BlockSpec(memory_space=pltpu.VMEM),
            pl.BlockSpec(memory_space=pltpu.SMEM),
        ],
        out_shape=[
            jax.ShapeDtypeStruct((T,), jnp.int32),
            jax.ShapeDtypeStruct((T,), jnp.int32),
            jax.ShapeDtypeStruct((T, 16), jnp.float32),
            jax.ShapeDtypeStruct((T, 16), jnp.float32),
            jax.ShapeDtypeStruct((1, 1), jnp.float32),
        ],
        interpret=interpret,
    )(scale, xf, W_gate)


def _experts_body(buf_ref, w1_ref, w2_ref, out_ref, acc_ref, *, nf):
    j = pl.program_id(1)

    @pl.when(j == 0)
    def _():
        acc_ref[...] = jnp.zeros_like(acc_ref)

    h = jnp.maximum(jnp.dot(buf_ref[...], w1_ref[0],
                            preferred_element_type=jnp.float32), 0.0)
    acc_ref[...] += jnp.dot(h, w2_ref[0],
                            preferred_element_type=jnp.float32)

    @pl.when(j == nf - 1)
    def _():
        out_ref[...] = acc_ref[...]


def _experts(buf, W1, W2, *, E, CP, D, DFF, FBLK=512, interpret=False):
    nf = DFF // FBLK
    body = functools.partial(_experts_body, nf=nf)
    return pl.pallas_call(
        body,
        grid=(E, nf),
        in_specs=[
            pl.BlockSpec((CP, D), lambda e, j: (e, 0)),
            pl.BlockSpec((1, D, FBLK), lambda e, j: (e, 0, j)),
            pl.BlockSpec((1, FBLK, D), lambda e, j: (e, j, 0)),
        ],
        out_specs=pl.BlockSpec((CP, D), lambda e, j: (e, 0)),
        out_shape=jax.ShapeDtypeStruct((E * CP, D), jnp.float32),
        scratch_shapes=[pltpu.VMEM((CP, D), jnp.float32)],
        compiler_params=pltpu.CompilerParams(
            dimension_semantics=("parallel", "arbitrary")),
        interpret=interpret,
    )(buf, W1, W2)


def _make_dispatch(T, D, NSLOT):
    info = plsc.get_sparse_core_info()
    NC, NS = info.num_cores, info.num_subcores
    NW = NC * NS
    per_w = T // NW
    CH = 64
    nch = per_w // CH
    mesh = plsc.VectorSubcoreMesh(core_axis_name="c", subcore_axis_name="s")

    @functools.partial(
        pl.kernel, mesh=mesh,
        out_type=jax.ShapeDtypeStruct((NSLOT, D), jnp.float32),
        scratch_types=[
            pltpu.VMEM((CH, D), jnp.float32),
            pltpu.VMEM((CH,), jnp.int32),
            pltpu.VMEM((CH,), jnp.int32),
            pltpu.SemaphoreType.DMA,
            pltpu.SemaphoreType.DMA,
        ],
    )
    def dispatch(x_hbm, s0_hbm, s1_hbm, buf_hbm,
                 rows_v, i0_v, i1_v, sem0, sem1):
        wid = lax.axis_index("s") * NC + lax.axis_index("c")
        for c in range(nch):
            base = wid * per_w + c * CH
            pltpu.sync_copy(x_hbm.at[pl.ds(base, CH)], rows_v)
            pltpu.sync_copy(s0_hbm.at[pl.ds(base, CH)], i0_v)
            pltpu.sync_copy(s1_hbm.at[pl.ds(base, CH)], i1_v)
            cp0 = pltpu.async_copy(rows_v, buf_hbm.at[i0_v], sem0)
            cp1 = pltpu.async_copy(rows_v, buf_hbm.at[i1_v], sem1)
            cp0.wait()
            cp1.wait()

    return dispatch


def _make_combine(T, D, NSLOT):
    info = plsc.get_sparse_core_info()
    NC, NS = info.num_cores, info.num_subcores
    NW = NC * NS
    per_w = T // NW
    CH = 32
    nch = per_w // CH
    nvec = D // 16
    mesh = plsc.VectorSubcoreMesh(core_axis_name="c", subcore_axis_name="s")

    @functools.partial(
        pl.kernel, mesh=mesh,
        out_type=jax.ShapeDtypeStruct((T, D), jnp.float32),
        scratch_types=[
            pltpu.VMEM((CH, D), jnp.float32),
            pltpu.VMEM((CH, D), jnp.float32),
            pltpu.VMEM((CH, D), jnp.float32),
            pltpu.VMEM((CH,), jnp.int32),
            pltpu.VMEM((CH,), jnp.int32),
            pltpu.VMEM((CH, 16), jnp.float32),
            pltpu.VMEM((CH, 16), jnp.float32),
            pltpu.SemaphoreType.DMA,
            pltpu.SemaphoreType.DMA,
        ],
    )
    def combine(out_hbm, s0_hbm, s1_hbm, w0_hbm, w1_hbm, y_hbm,
                r0_v, r1_v, y_v, i0_v, i1_v, wv0, wv1, sem0, sem1):
        wid = lax.axis_index("s") * NC + lax.axis_index("c")
        for c in range(nch):
            base = wid * per_w + c * CH
            pltpu.sync_copy(s0_hbm.at[pl.ds(base, CH)], i0_v)
            pltpu.sync_copy(s1_hbm.at[pl.ds(base, CH)], i1_v)
            pltpu.sync_copy(w0_hbm.at[pl.ds(base, CH)], wv0)
            pltpu.sync_copy(w1_hbm.at[pl.ds(base, CH)], wv1)
            g0 = pltpu.async_copy(out_hbm.at[i0_v], r0_v, sem0)
            g1 = pltpu.async_copy(out_hbm.at[i1_v], r1_v, sem1)
            g0.wait()
            g1.wait()

            def row(i, _):
                a = wv0[i]
                b = wv1[i]

                def col(j, __):
                    sl = pl.ds(j * 16, 16)
                    y_v[i, sl] = a * r0_v[i, sl] + b * r1_v[i, sl]
                    return __

                return lax.fori_loop(0, nvec, col, _, unroll=8)

            lax.fori_loop(0, CH, row, 0)
            pltpu.sync_copy(y_v, y_hbm.at[pl.ds(base, CH)])

    return combine


def kernel(x, W_gate, W1, W2, k):
    B, S, D = x.shape
    E = W_gate.shape[1]
    DFF = W1.shape[2]
    T = B * S
    C = math.ceil(CAPACITY_FACTOR * T / E)
    CP = ((C + 8) + 7) // 8 * 8
    NSLOT = E * CP

    xf = x.reshape(T, D)
    scale = (jnp.asarray(k, jnp.float32) / K_TOP).reshape(1, 1)

    slot0, slot1, w0b, w1b, aux = _route(xf, W_gate, scale,
                                         T=T, E=E, C=C, CP=CP)
    buf = _make_dispatch(T, D, NSLOT)(xf, slot0, slot1)
    out = _experts(buf, W1, W2, E=E, CP=CP, D=D, DFF=DFF)
    y = _make_combine(T, D, NSLOT)(out, slot0, slot1, w0b, w1b)
    return y.reshape(B, S, D), aux.reshape(())


# speedup vs baseline: 1.6839x; 1.0735x over previous
import functools
import math

import jax
import jax.numpy as jnp
from jax import lax
from jax.experimental import pallas as pl
from jax.experimental.pallas import tpu as pltpu
from jax.experimental.pallas import tpu_sc as plsc

CAPACITY_FACTOR = 1.25
K_TOP = 2


def _route_body(scale_ref, x_ref, wg_ref,
                slot0_ref, slot1_ref, w0b_ref, w1b_ref, aux_ref,
                *, T, E, C, CP, BLK):
    scale = scale_ref[0, 0]
    logits = jnp.dot(x_ref[...], wg_ref[...],
                     preferred_element_type=jnp.float32) * scale

    iota_e = lax.broadcasted_iota(jnp.int32, (T, E), 1)
    m1 = jnp.max(logits, axis=1, keepdims=True)
    a1 = jnp.min(jnp.where(logits == m1, iota_e, E), axis=1, keepdims=True)
    oh0 = (iota_e == a1)
    l2 = jnp.where(oh0, jnp.float32(-1e30), logits)
    m2 = jnp.max(l2, axis=1, keepdims=True)
    a2 = jnp.min(jnp.where(l2 == m2, iota_e, E), axis=1, keepdims=True)
    oh1 = (iota_e == a2)

    z = jnp.exp(m2 - m1)
    p0 = 1.0 / (1.0 + z)
    p1 = z / (1.0 + z)

    oh0f = oh0.astype(jnp.float32)
    oh1f = oh1.astype(jnp.float32)
    ohsum = oh0f + oh1f

    nb = T // BLK
    tril = (lax.broadcasted_iota(jnp.int32, (BLK, BLK), 0)
            > lax.broadcasted_iota(jnp.int32, (BLK, BLK), 1)).astype(jnp.float32)
    carry = jnp.zeros((1, E), jnp.float32)
    pieces = []
    for b in range(nb):
        blk = ohsum[b * BLK:(b + 1) * BLK, :]
        local = jnp.dot(tril, blk, preferred_element_type=jnp.float32)
        pieces.append(local + carry)
        carry = carry + jnp.sum(blk, axis=0, keepdims=True)
    cumexcl = jnp.concatenate(pieces, axis=0)

    cnt0 = jnp.sum(oh0f * cumexcl, axis=1, keepdims=True) + 1.0
    cnt1 = jnp.sum(oh1f * cumexcl, axis=1, keepdims=True) + 1.0
    keep0 = cnt0 <= C
    keep1 = cnt1 <= C
    pos0 = cnt0.astype(jnp.int32) - 1
    pos1 = cnt1.astype(jnp.int32) - 1

    p0k = p0 * keep0.astype(jnp.float32)
    p1k = p1 * keep1.astype(jnp.float32)
    denom = p0k + p1k + 1e-9
    w0 = jnp.where(keep0, p0k / denom, 0.0)
    w1 = jnp.where(keep1, p1k / denom, 0.0)

    slot0 = jnp.where(keep0, a1 * CP + pos0, a1 * CP + C)
    slot1 = jnp.where(keep1, a2 * CP + pos1, a2 * CP + C)

    slot0_ref[...] = slot0[:, 0]
    slot1_ref[...] = slot1[:, 0]
    w0b_ref[...] = jnp.broadcast_to(w0, (T, 16))
    w1b_ref[...] = jnp.broadcast_to(w1, (T, 16))

    k0f = keep0.astype(jnp.float32)
    k1f = keep1.astype(jnp.float32)
    tokens_per_e = jnp.sum(k0f * oh0f + k1f * oh1f, axis=0, keepdims=True)
    importance = jnp.sum(w0 * oh0f + w1 * oh1f, axis=0, keepdims=True)
    tf = tokens_per_e / (jnp.sum(tokens_per_e) + 1e-9)
    imf = importance / (jnp.sum(importance) + 1e-9)
    aux_ref[0, 0] = jnp.sum(tf * imf) * E


def _route(xf, W_gate, scale, *, T, E, C, CP, interpret=False):
    body = functools.partial(_route_body, T=T, E=E, C=C, CP=CP, BLK=128)
    return pl.pallas_call(
        body,
        in_specs=[
            pl.BlockSpec(memory_space=pltpu.SMEM),
            pl.BlockSpec(memory_space=pltpu.VMEM),
            pl.BlockSpec(memory_space=pltpu.VMEM),
        ],
        out_specs=[
            pl.BlockSpec(memory_space=pltpu.VMEM),
            pl.BlockSpec(memory_space=pltpu.VMEM),
            pl.BlockSpec(memory_space=pltpu.VMEM),
            pl.BlockSpec(memory_space=pltpu.VMEM),
            pl.BlockSpec(memory_space=pltpu.SMEM),
        ],
        out_shape=[
            jax.ShapeDtypeStruct((T,), jnp.int32),
            jax.ShapeDtypeStruct((T,), jnp.int32),
            jax.ShapeDtypeStruct((T, 16), jnp.float32),
            jax.ShapeDtypeStruct((T, 16), jnp.float32),
            jax.ShapeDtypeStruct((1, 1), jnp.float32),
        ],
        interpret=interpret,
    )(scale, xf, W_gate)


def _experts_body(buf_ref, w1_ref, w2_ref, out_ref, acc_ref, *, nf):
    j = pl.program_id(1)

    @pl.when(j == 0)
    def _():
        acc_ref[...] = jnp.zeros_like(acc_ref)

    h = jnp.maximum(jnp.dot(buf_ref[...], w1_ref[0],
                            preferred_element_type=jnp.float32), 0.0)
    acc_ref[...] += jnp.dot(h, w2_ref[0],
                            preferred_element_type=jnp.float32)

    @pl.when(j == nf - 1)
    def _():
        out_ref[...] = acc_ref[...]


def _experts(buf, W1, W2, *, E, CP, D, DFF, FBLK=512, interpret=False):
    nf = DFF // FBLK
    body = functools.partial(_experts_body, nf=nf)
    return pl.pallas_call(
        body,
        grid=(E, nf),
        in_specs=[
            pl.BlockSpec((CP, D), lambda e, j: (e, 0)),
            pl.BlockSpec((1, D, FBLK), lambda e, j: (e, 0, j)),
            pl.BlockSpec((1, FBLK, D), lambda e, j: (e, j, 0)),
        ],
        out_specs=pl.BlockSpec((CP, D), lambda e, j: (e, 0)),
        out_shape=jax.ShapeDtypeStruct((E * CP, D), jnp.float32),
        scratch_shapes=[pltpu.VMEM((CP, D), jnp.float32)],
        compiler_params=pltpu.CompilerParams(
            dimension_semantics=("parallel", "arbitrary")),
        interpret=interpret,
    )(buf, W1, W2)


def _make_dispatch(T, D, NSLOT):
    info = plsc.get_sparse_core_info()
    NC, NS = info.num_cores, info.num_subcores
    NW = NC * NS
    per_w = T // NW
    CH = 32
    nch = per_w // CH
    mesh = plsc.VectorSubcoreMesh(core_axis_name="c", subcore_axis_name="s")

    @functools.partial(
        pl.kernel, mesh=mesh,
        out_type=jax.ShapeDtypeStruct((NSLOT, D), jnp.float32),
        scratch_types=[
            pltpu.VMEM((CH, D), jnp.float32),
            pltpu.VMEM((CH, D), jnp.float32),
            pltpu.VMEM((nch, CH), jnp.int32),
            pltpu.VMEM((nch, CH), jnp.int32),
            pltpu.SemaphoreType.DMA,
            pltpu.SemaphoreType.DMA,
            pltpu.SemaphoreType.DMA,
            pltpu.SemaphoreType.DMA,
            pltpu.SemaphoreType.DMA,
            pltpu.SemaphoreType.DMA,
        ],
    )
    def dispatch(x_hbm, s0_hbm, s1_hbm, buf_hbm,
                 rb0, rb1, i0_v, i1_v, sl0, sl1, ss0a, ss0b, ss1a, ss1b):
        wid = lax.axis_index("s") * NC + lax.axis_index("c")
        pltpu.sync_copy(s0_hbm.at[wid], i0_v)
        pltpu.sync_copy(s1_hbm.at[wid], i1_v)
        rbufs = (rb0, rb1)
        slsem = (sl0, sl1)
        s0sem = (ss0a, ss0b)
        s1sem = (ss1a, ss1b)
        loads = [None] * nch
        scat = [None] * nch
        loads[0] = pltpu.async_copy(
            x_hbm.at[pl.ds(wid * per_w, CH)], rbufs[0], slsem[0])
        for c in range(nch):
            p = c % 2
            loads[c].wait()
            scat[c] = (
                pltpu.async_copy(rbufs[p], buf_hbm.at[i0_v.at[c]], s0sem[p]),
                pltpu.async_copy(rbufs[p], buf_hbm.at[i1_v.at[c]], s1sem[p]),
            )
            if c + 1 < nch:
                if c >= 1:
                    scat[c - 1][0].wait()
                    scat[c - 1][1].wait()
                loads[c + 1] = pltpu.async_copy(
                    x_hbm.at[pl.ds(wid * per_w + (c + 1) * CH, CH)],
                    rbufs[(c + 1) % 2], slsem[(c + 1) % 2])
        if nch >= 2:
            scat[nch - 2][0].wait()
            scat[nch - 2][1].wait()
        scat[nch - 1][0].wait()
        scat[nch - 1][1].wait()

    return dispatch


def _make_combine(T, D, NSLOT):
    info = plsc.get_sparse_core_info()
    NC, NS = info.num_cores, info.num_subcores
    NW = NC * NS
    per_w = T // NW
    CH = 16
    nch = per_w // CH
    nvec = D // 16
    mesh = plsc.VectorSubcoreMesh(core_axis_name="c", subcore_axis_name="s")

    @functools.partial(
        pl.kernel, mesh=mesh,
        out_type=jax.ShapeDtypeStruct((T, D), jnp.float32),
        scratch_types=[
            pltpu.VMEM((CH, D), jnp.float32),
            pltpu.VMEM((CH, D), jnp.float32),
            pltpu.VMEM((CH, D), jnp.float32),
            pltpu.VMEM((CH, D), jnp.float32),
            pltpu.VMEM((nch, CH), jnp.int32),
            pltpu.VMEM((nch, CH), jnp.int32),
            pltpu.VMEM((per_w, 16), jnp.float32),
            pltpu.VMEM((per_w, 16), jnp.float32),
            pltpu.SemaphoreType.DMA,
            pltpu.SemaphoreType.DMA,
            pltpu.SemaphoreType.DMA,
            pltpu.SemaphoreType.DMA,
            pltpu.SemaphoreType.DMA,
            pltpu.SemaphoreType.DMA,
        ],
    )
    def combine(out_hbm, s0_hbm, s1_hbm, w0_hbm, w1_hbm, y_hbm,
                r0a, r0b, r1a, r1b, i0_v, i1_v, wv0, wv1,
                sg0a, sg0b, sg1a, sg1b, sya, syb):
        wid = lax.axis_index("s") * NC + lax.axis_index("c")
        pltpu.sync_copy(s0_hbm.at[wid], i0_v)
        pltpu.sync_copy(s1_hbm.at[wid], i1_v)
        pltpu.sync_copy(w0_hbm.at[wid], wv0)
        pltpu.sync_copy(w1_hbm.at[wid], wv1)
        r0 = (r0a, r0b)
        r1 = (r1a, r1b)
        g0sem = (sg0a, sg0b)
        g1sem = (sg1a, sg1b)
        ysem = (sya, syb)
        gath = [None] * nch
        ystore = [None] * nch
        gath[0] = (
            pltpu.async_copy(out_hbm.at[i0_v.at[0]], r0[0], g0sem[0]),
            pltpu.async_copy(out_hbm.at[i1_v.at[0]], r1[0], g1sem[0]),
        )
        for c in range(nch):
            p = c % 2
            gath[c][0].wait()
            gath[c][1].wait()
            if c + 1 < nch:
                q = (c + 1) % 2
                if c >= 1:
                    ystore[c - 1].wait()
                gath[c + 1] = (
                    pltpu.async_copy(out_hbm.at[i0_v.at[c + 1]], r0[q],
                                     g0sem[q]),
                    pltpu.async_copy(out_hbm.at[i1_v.at[c + 1]], r1[q],
                                     g1sem[q]),
                )
            y_v, b_v = r0[p], r1[p]

            def row(i, _, y_v=y_v, b_v=b_v, c=c):
                a = wv0[c * CH + i]
                b = wv1[c * CH + i]

                def col(j, __):
                    sl = pl.ds(j * 16, 16)
                    y_v[i, sl] = a * y_v[i, sl] + b * b_v[i, sl]
                    return __

                return lax.fori_loop(0, nvec, col, _, unroll=8)

            lax.fori_loop(0, CH, row, 0)
            ystore[c] = pltpu.async_copy(
                y_v, y_hbm.at[pl.ds(wid * per_w + c * CH, CH)], ysem[p])
        if nch >= 2:
            ystore[nch - 2].wait()
        ystore[nch - 1].wait()

    return combine


def kernel(x, W_gate, W1, W2, k):
    B, S, D = x.shape
    E = W_gate.shape[1]
    DFF = W1.shape[2]
    T = B * S
    C = math.ceil(CAPACITY_FACTOR * T / E)
    CP = ((C + 8) + 7) // 8 * 8
    NSLOT = E * CP

    xf = x.reshape(T, D)
    scale = (jnp.asarray(k, jnp.float32) / K_TOP).reshape(1, 1)

    slot0, slot1, w0b, w1b, aux = _route(xf, W_gate, scale,
                                         T=T, E=E, C=C, CP=CP)
    NW = 32
    s0d = slot0.reshape(NW, 4, 32)
    s1d = slot1.reshape(NW, 4, 32)
    s0c = slot0.reshape(NW, 8, 16)
    s1c = slot1.reshape(NW, 8, 16)
    w0r = w0b.reshape(NW, T // NW, 16)
    w1r = w1b.reshape(NW, T // NW, 16)
    buf = _make_dispatch(T, D, NSLOT)(xf, s0d, s1d)
    out = _experts(buf, W1, W2, E=E, CP=CP, D=D, DFF=DFF)
    y = _make_combine(T, D, NSLOT)(out, s0c, s1c, w0r, w1r)
    return y.reshape(B, S, D), aux.reshape(())
